# baseline (speedup 1.0000x reference)
import functools
import math

import jax
import jax.numpy as jnp
from jax.experimental import pallas as pl
from jax.experimental.pallas import tpu as pltpu
from jax.experimental.pallas import tpu_sc as plsc

_B, _N, _D, _E, _KE = 2, 2048, 1024, 16, 8
_NT = 512
_NNT = _N // _NT

_DN0 = (((0,), (0,)), ((), ()))


def _assign_tile_t(idxT, wT):
    nt = idxT.shape[-1]
    iota_e = jax.lax.broadcasted_iota(jnp.int32, (_E, nt), 0)
    acc = jnp.zeros((_E, nt), jnp.float32)
    for k in range(_KE):
        acc = acc + jnp.where(idxT[k:k + 1, :] == iota_e, wT[k:k + 1, :], 0.0)
    return acc


_SC_CH = 256


def _sc_assign_body(idxAT_hbm, wAT_hbm, idxBT_hbm, wBT_hbm,
                    outA_hbm, outB_hbm, idx_v, w_v, acc_v):
    nc = 2
    wid = jax.lax.axis_index("s") * nc + jax.lax.axis_index("c")
    b = wid // 16
    side = (wid // 8) % 2
    base = (wid % 8) * _SC_CH

    def run(idx_hbm, w_hbm, out_hbm):
        pltpu.sync_copy(idx_hbm.at[b, :, pl.ds(base, _SC_CH)], idx_v)
        pltpu.sync_copy(w_hbm.at[b, :, pl.ds(base, _SC_CH)], w_v)
        zero = jnp.zeros((16,), jnp.float32)

        def gbody(g, carry):
            gsl = pl.ds(g * 16, 16)
            acc = [zero] * _E
            for k in range(_KE):
                iv = idx_v[k, gsl]
                wv = w_v[k, gsl]
                for e in range(_E):
                    acc[e] = acc[e] + jnp.where(iv == e, wv, 0.0)
            for e in range(_E):
                acc_v[e, gsl] = acc[e]
            return carry

        jax.lax.fori_loop(0, _SC_CH // 16, gbody, 0)
        pltpu.sync_copy(acc_v, out_hbm.at[b, :, pl.ds(base, _SC_CH)])

    @pl.when(side == 0)
    def _():
        run(idxAT_hbm, wAT_hbm, outA_hbm)

    @pl.when(side == 1)
    def _():
        run(idxBT_hbm, wBT_hbm, outB_hbm)


def _sc_assign(idxAT, wAT, idxBT, wBT):
    k = functools.partial(
        pl.kernel,
        mesh=plsc.VectorSubcoreMesh(core_axis_name="c", subcore_axis_name="s"),
        compiler_params=pltpu.CompilerParams(use_tc_tiling_on_sc=False),
        out_type=[jax.ShapeDtypeStruct((_B, _E, _N), jnp.float32),
                  jax.ShapeDtypeStruct((_B, _E, _N), jnp.float32)],
        scratch_types=[pltpu.VMEM((_KE, _SC_CH), jnp.int32),
                       pltpu.VMEM((_KE, _SC_CH), jnp.float32),
                       pltpu.VMEM((_E, _SC_CH), jnp.float32)],
    )(_sc_assign_body)
    return k(idxAT, wAT, idxBT, wBT)


def _gelu_exact(x):
    return 0.5 * x * (1.0 + jax.lax.erf(x * (1.0 / math.sqrt(2.0))))


def _body(atA_ref, atB_ref, xA_ref, xB_ref,
          wb2a_hbm, wa2b_hbm, wgA_hbm, wgB_hbm,
          bb2a_ref, ba2b_ref, bgA_ref, bgB_ref,
          outA_ref, outB_ref,
          xAs, xBs, AtS, BtS, heA_s, heB_s,
          heAt_s, heBt_s, mA_s, mB_s, wgAtop_s, wgBtop_s,
          wb2a_v, wa2b_v, botA_v, botB_v,
          sem0, sem1, sem2, sem3):
    i = pl.program_id(0)
    n = pl.program_id(1)
    nsl = pl.ds(n * _NT, _NT)

    c_b2a = pltpu.make_async_copy(wb2a_hbm, wb2a_v, sem0)
    c_a2b = pltpu.make_async_copy(wa2b_hbm, wa2b_v, sem1)
    c_botA = pltpu.make_async_copy(wgA_hbm.at[pl.ds(_D, _D), :], botA_v, sem2)
    c_botB = pltpu.make_async_copy(wgB_hbm.at[pl.ds(_D, _D), :], botB_v, sem3)
    c_topA = pltpu.make_async_copy(wgA_hbm.at[pl.ds(0, _D), :], botA_v, sem2)
    c_topB = pltpu.make_async_copy(wgB_hbm.at[pl.ds(0, _D), :], botB_v, sem3)

    @pl.when(jnp.logical_and(i == 0, n == 0))
    def _():
        c_topA.start()
        c_topB.start()
        c_b2a.start()
        c_a2b.start()

    @pl.when(jnp.logical_and(i == 0, n == 1))
    def _():
        c_topA.wait()
        wgAtop_s[...] = botA_v[...].astype(jnp.bfloat16)
        c_botA.start()

    @pl.when(jnp.logical_and(i == 0, n == 2))
    def _():
        c_topB.wait()
        wgBtop_s[...] = botB_v[...].astype(jnp.bfloat16)
        c_botB.start()

    @pl.when(i < _B)
    def _():
        bb = pl.ds(i % 2, 1)
        At = atA_ref[0]
        Bt = atB_ref[0]
        AtS[bb, :, nsl] = At[None]
        BtS[bb, :, nsl] = Bt[None]
        xa = xA_ref[0]
        xb = xB_ref[0]
        xAs[bb, nsl, :] = xa.astype(jnp.bfloat16)[None]
        xBs[bb, nsl, :] = xb.astype(jnp.bfloat16)[None]
        heA = jnp.dot(At, xa, preferred_element_type=jnp.float32)
        heB = jnp.dot(Bt, xb, preferred_element_type=jnp.float32)

        @pl.when(n == 0)
        def _():
            heA_s[bb] = heA[None]
            heB_s[bb] = heB[None]

        @pl.when(n != 0)
        def _():
            heA_s[bb] += heA[None]
            heB_s[bb] += heB[None]

    @pl.when(i >= 1)
    def _():
        bb = pl.ds((i - 1) % 2, 1)

        @pl.when(n == 0)
        def _():
            @pl.when(i == 1)
            def _():
                c_b2a.wait()
                c_a2b.wait()
                c_botA.wait()
                c_botB.wait()

            heAt = _gelu_exact(
                jnp.dot(heA_s[bb][0], wb2a_v[...],
                        preferred_element_type=jnp.float32) + bb2a_ref[...])
            heBt = _gelu_exact(
                jnp.dot(heB_s[bb][0], wa2b_v[...],
                        preferred_element_type=jnp.float32) + ba2b_ref[...])
            heAt_s[...] = heAt
            heBt_s[...] = heBt
            mA_s[...] = jnp.dot(heBt, botA_v[...],
                                preferred_element_type=jnp.float32)
            mB_s[...] = jnp.dot(heAt, botB_v[...],
                                preferred_element_type=jnp.float32)

        At = AtS[bb, :, nsl][0]
        Bt = BtS[bb, :, nsl][0]

        xa = xAs[bb, nsl, :][0]
        preA = (jnp.dot(xa, wgAtop_s[...], preferred_element_type=jnp.float32)
                + jax.lax.dot_general(At, mA_s[...], _DN0,
                                      preferred_element_type=jnp.float32)
                + bgA_ref[...])
        gA = jax.nn.sigmoid(preA)
        xAfromB = jax.lax.dot_general(At, heBt_s[...], _DN0,
                                      preferred_element_type=jnp.float32)
        outA_ref[0] = gA * xa.astype(jnp.float32) + (1.0 - gA) * xAfromB

        xb = xBs[bb, nsl, :][0]
        preB = (jnp.dot(xb, wgBtop_s[...], preferred_element_type=jnp.float32)
                + jax.lax.dot_general(Bt, mB_s[...], _DN0,
                                      preferred_element_type=jnp.float32)
                + bgB_ref[...])
        gB = jax.nn.sigmoid(preB)
        xBfromA = jax.lax.dot_general(Bt, heAt_s[...], _DN0,
                                      preferred_element_type=jnp.float32)
        outB_ref[0] = gB * xb.astype(jnp.float32) + (1.0 - gB) * xBfromA


def kernel(X_A, X_B, idxA, wA, idxB, wB, E, W_A2B, b_A2B, W_B2A, b_B2A,
           Wg_A, bg_A, Wg_B, bg_B):
    del E
    f32 = jnp.float32
    bf16 = jnp.bfloat16
    last = _NNT - 1
    lastb = _B - 1

    idxAT = jnp.swapaxes(idxA, 1, 2)
    wAT = jnp.swapaxes(wA, 1, 2)
    idxBT = jnp.swapaxes(idxB, 1, 2)
    wBT = jnp.swapaxes(wB, 1, 2)

    at_A, at_B = _sc_assign(idxAT, wAT, idxBT, wBT)

    def p0_map3(i, n):
        done = (i >= _B).astype(jnp.int32)
        return (jnp.minimum(i, lastb), 0, n + (last - n) * done)

    def p0_mapx(i, n):
        done = (i >= _B).astype(jnp.int32)
        return (jnp.minimum(i, lastb), n + (last - n) * done, 0)

    def p1_mapx(i, n):
        act = (i >= 1).astype(jnp.int32)
        return (jnp.maximum(i - 1, 0), n * act, 0)

    at_spec = pl.BlockSpec((1, _E, _NT), p0_map3)
    x_spec = pl.BlockSpec((1, _NT, _D), p0_mapx)
    out_spec = pl.BlockSpec((1, _NT, _D), p1_mapx)
    any_spec = pl.BlockSpec(memory_space=pl.ANY)
    bias_spec = pl.BlockSpec((1, _D), lambda i, n: (0, 0))

    scr_e = pltpu.VMEM((_E, _D), f32)

    out_A, out_B = pl.pallas_call(
        _body,
        grid=(_B + 1, _NNT),
        in_specs=[at_spec, at_spec,
                  x_spec, x_spec,
                  any_spec, any_spec, any_spec, any_spec,
                  bias_spec, bias_spec, bias_spec, bias_spec],
        out_specs=[out_spec, out_spec],
        out_shape=[jax.ShapeDtypeStruct((_B, _N, _D), f32),
                   jax.ShapeDtypeStruct((_B, _N, _D), f32)],
        scratch_shapes=[pltpu.VMEM((2, _N, _D), bf16),
                        pltpu.VMEM((2, _N, _D), bf16),
                        pltpu.VMEM((2, _E, _N), f32),
                        pltpu.VMEM((2, _E, _N), f32),
                        pltpu.VMEM((2, _E, _D), f32),
                        pltpu.VMEM((2, _E, _D), f32),
                        scr_e, scr_e, scr_e, scr_e,
                        pltpu.VMEM((_D, _D), bf16), pltpu.VMEM((_D, _D), bf16),
                        pltpu.VMEM((_D, _D), f32), pltpu.VMEM((_D, _D), f32),
                        pltpu.VMEM((_D, _D), f32), pltpu.VMEM((_D, _D), f32),
                        pltpu.SemaphoreType.DMA, pltpu.SemaphoreType.DMA,
                        pltpu.SemaphoreType.DMA, pltpu.SemaphoreType.DMA],
    )(at_A, at_B, X_A, X_B,
      W_B2A, W_A2B, Wg_A, Wg_B,
      b_B2A.reshape(1, _D), b_A2B.reshape(1, _D),
      bg_A.reshape(1, _D), bg_B.reshape(1, _D))

    return (out_A, out_B)


# speedup vs baseline: 1.5721x; 1.5721x over previous
import math

import jax
import jax.numpy as jnp
from jax.experimental import pallas as pl
from jax.experimental.pallas import tpu as pltpu

_B, _N, _D, _E, _KE = 2, 2048, 1024, 16, 8
_NT = 512
_NNT = _N // _NT

_DN0 = (((0,), (0,)), ((), ()))


def _assign_tile_t(idxT, wT):
    nt = idxT.shape[-1]
    iota_e = jax.lax.broadcasted_iota(jnp.int32, (_E, nt), 0)
    acc = jnp.zeros((_E, nt), jnp.float32)
    for k in range(_KE):
        acc = acc + jnp.where(idxT[k:k + 1, :] == iota_e, wT[k:k + 1, :], 0.0)
    return acc


def _gelu_exact(x):
    return 0.5 * x * (1.0 + jax.lax.erf(x * (1.0 / math.sqrt(2.0))))


def _body(idxAT_ref, wAT_ref, idxBT_ref, wBT_ref, xA_ref, xB_ref,
          wb2a_hbm, wa2b_hbm, wgA_hbm, wgB_hbm,
          bb2a_ref, ba2b_ref, bgA_ref, bgB_ref,
          outA_ref, outB_ref,
          xAs, xBs, AtS, BtS, heA_s, heB_s,
          heAt_s, heBt_s, mA_s, mB_s, wgAtop_s, wgBtop_s,
          wb2a_v, wa2b_v, botA_v, botB_v,
          sem0, sem1, sem2, sem3):
    i = pl.program_id(0)
    n = pl.program_id(1)
    nsl = pl.ds(n * _NT, _NT)

    c_b2a = pltpu.make_async_copy(wb2a_hbm, wb2a_v, sem0)
    c_a2b = pltpu.make_async_copy(wa2b_hbm, wa2b_v, sem1)
    c_botA = pltpu.make_async_copy(wgA_hbm.at[pl.ds(_D, _D), :], botA_v, sem2)
    c_botB = pltpu.make_async_copy(wgB_hbm.at[pl.ds(_D, _D), :], botB_v, sem3)
    c_topA = pltpu.make_async_copy(wgA_hbm.at[pl.ds(0, _D), :], botA_v, sem2)
    c_topB = pltpu.make_async_copy(wgB_hbm.at[pl.ds(0, _D), :], botB_v, sem3)

    @pl.when(jnp.logical_and(i == 0, n == 0))
    def _():
        c_topA.start()
        c_topB.start()
        c_b2a.start()
        c_a2b.start()

    @pl.when(jnp.logical_and(i == 0, n == 1))
    def _():
        c_topA.wait()
        wgAtop_s[...] = botA_v[...].astype(jnp.bfloat16)
        c_botA.start()

    @pl.when(jnp.logical_and(i == 0, n == 2))
    def _():
        c_topB.wait()
        wgBtop_s[...] = botB_v[...].astype(jnp.bfloat16)
        c_botB.start()

    @pl.when(i < _B)
    def _():
        bb = pl.ds(i % 2, 1)
        At = _assign_tile_t(idxAT_ref[0], wAT_ref[0])
        Bt = _assign_tile_t(idxBT_ref[0], wBT_ref[0])
        AtS[bb, :, nsl] = At[None]
        BtS[bb, :, nsl] = Bt[None]
        xa = xA_ref[0]
        xb = xB_ref[0]
        xAs[bb, nsl, :] = xa.astype(jnp.bfloat16)[None]
        xBs[bb, nsl, :] = xb.astype(jnp.bfloat16)[None]
        heA = jnp.dot(At, xa, preferred_element_type=jnp.float32)
        heB = jnp.dot(Bt, xb, preferred_element_type=jnp.float32)

        @pl.when(n == 0)
        def _():
            heA_s[bb] = heA[None]
            heB_s[bb] = heB[None]

        @pl.when(n != 0)
        def _():
            heA_s[bb] += heA[None]
            heB_s[bb] += heB[None]

    @pl.when(i >= 1)
    def _():
        bb = pl.ds((i - 1) % 2, 1)

        @pl.when(n == 0)
        def _():
            @pl.when(i == 1)
            def _():
                c_b2a.wait()
                c_a2b.wait()
                c_botA.wait()
                c_botB.wait()

            heAt = _gelu_exact(
                jnp.dot(heA_s[bb][0], wb2a_v[...],
                        preferred_element_type=jnp.float32) + bb2a_ref[...])
            heBt = _gelu_exact(
                jnp.dot(heB_s[bb][0], wa2b_v[...],
                        preferred_element_type=jnp.float32) + ba2b_ref[...])
            heAt_s[...] = heAt
            heBt_s[...] = heBt
            mA_s[...] = jnp.dot(heBt, botA_v[...],
                                preferred_element_type=jnp.float32)
            mB_s[...] = jnp.dot(heAt, botB_v[...],
                                preferred_element_type=jnp.float32)

        At = AtS[bb, :, nsl][0]
        Bt = BtS[bb, :, nsl][0]

        xa = xAs[bb, nsl, :][0]
        preA = (jnp.dot(xa, wgAtop_s[...], preferred_element_type=jnp.float32)
                + jax.lax.dot_general(At, mA_s[...], _DN0,
                                      preferred_element_type=jnp.float32)
                + bgA_ref[...])
        gA = jax.nn.sigmoid(preA)
        xAfromB = jax.lax.dot_general(At, heBt_s[...], _DN0,
                                      preferred_element_type=jnp.float32)
        outA_ref[0] = gA * xa.astype(jnp.float32) + (1.0 - gA) * xAfromB

        xb = xBs[bb, nsl, :][0]
        preB = (jnp.dot(xb, wgBtop_s[...], preferred_element_type=jnp.float32)
                + jax.lax.dot_general(Bt, mB_s[...], _DN0,
                                      preferred_element_type=jnp.float32)
                + bgB_ref[...])
        gB = jax.nn.sigmoid(preB)
        xBfromA = jax.lax.dot_general(Bt, heAt_s[...], _DN0,
                                      preferred_element_type=jnp.float32)
        outB_ref[0] = gB * xb.astype(jnp.float32) + (1.0 - gB) * xBfromA


def kernel(X_A, X_B, idxA, wA, idxB, wB, E, W_A2B, b_A2B, W_B2A, b_B2A,
           Wg_A, bg_A, Wg_B, bg_B):
    del E
    f32 = jnp.float32
    bf16 = jnp.bfloat16
    last = _NNT - 1
    lastb = _B - 1

    idxAT = jnp.swapaxes(idxA, 1, 2)
    wAT = jnp.swapaxes(wA, 1, 2)
    idxBT = jnp.swapaxes(idxB, 1, 2)
    wBT = jnp.swapaxes(wB, 1, 2)

    def p0_map3(i, n):
        done = (i >= _B).astype(jnp.int32)
        return (jnp.minimum(i, lastb), 0, n + (last - n) * done)

    def p0_mapx(i, n):
        done = (i >= _B).astype(jnp.int32)
        return (jnp.minimum(i, lastb), n + (last - n) * done, 0)

    def p1_mapx(i, n):
        act = (i >= 1).astype(jnp.int32)
        return (jnp.maximum(i - 1, 0), n * act, 0)

    idxt_spec = pl.BlockSpec((1, _KE, _NT), p0_map3)
    x_spec = pl.BlockSpec((1, _NT, _D), p0_mapx)
    out_spec = pl.BlockSpec((1, _NT, _D), p1_mapx)
    any_spec = pl.BlockSpec(memory_space=pl.ANY)
    bias_spec = pl.BlockSpec((1, _D), lambda i, n: (0, 0))

    scr_e = pltpu.VMEM((_E, _D), f32)

    out_A, out_B = pl.pallas_call(
        _body,
        grid=(_B + 1, _NNT),
        in_specs=[idxt_spec, idxt_spec, idxt_spec, idxt_spec,
                  x_spec, x_spec,
                  any_spec, any_spec, any_spec, any_spec,
                  bias_spec, bias_spec, bias_spec, bias_spec],
        out_specs=[out_spec, out_spec],
        out_shape=[jax.ShapeDtypeStruct((_B, _N, _D), f32),
                   jax.ShapeDtypeStruct((_B, _N, _D), f32)],
        scratch_shapes=[pltpu.VMEM((2, _N, _D), bf16),
                        pltpu.VMEM((2, _N, _D), bf16),
                        pltpu.VMEM((2, _E, _N), f32),
                        pltpu.VMEM((2, _E, _N), f32),
                        pltpu.VMEM((2, _E, _D), f32),
                        pltpu.VMEM((2, _E, _D), f32),
                        scr_e, scr_e, scr_e, scr_e,
                        pltpu.VMEM((_D, _D), bf16), pltpu.VMEM((_D, _D), bf16),
                        pltpu.VMEM((_D, _D), f32), pltpu.VMEM((_D, _D), f32),
                        pltpu.VMEM((_D, _D), f32), pltpu.VMEM((_D, _D), f32),
                        pltpu.SemaphoreType.DMA, pltpu.SemaphoreType.DMA,
                        pltpu.SemaphoreType.DMA, pltpu.SemaphoreType.DMA],
    )(idxAT, wAT, idxBT, wBT, X_A, X_B,
      W_B2A, W_A2B, Wg_A, Wg_B,
      b_B2A.reshape(1, _D), b_A2B.reshape(1, _D),
      bg_A.reshape(1, _D), bg_B.reshape(1, _D))

    return (out_A, out_B)
